# baseline (speedup 1.0000x reference)
import functools

import jax
import jax.numpy as jnp
from jax import lax
from jax.experimental import pallas as pl
from jax.experimental.pallas import tpu as pltpu
from jax.experimental.pallas import tpu_sc as plsc

N = 10000
NP = 10240
E = 320000
NC = 2
NS = 16
NT = NC * NS
CH = 125
NCHUNK = 80
RPT = NP // NS
D = 64
NBUF = 4

_mesh = plsc.VectorSubcoreMesh(
    core_axis_name="c", subcore_axis_name="s", num_cores=NC, num_subcores=NS)
_sc_params = pltpu.CompilerParams(use_tc_tiling_on_sc=False)

_ZCHUNKS = [(o, min(CH, RPT - o)) for o in range(0, RPT, CH)]


def _fill_rows(buf, rows, cols, value):
    vec = jnp.full((16,), value, jnp.float32)

    def body(r, carry):
        for k in range(cols // 16):
            buf[r, pl.ds(k * 16, 16)] = vec
        return carry

    lax.fori_loop(0, rows, body, 0)


def _deg_body(se_hbm, deg_hbm, idx_v, ones_v, hist_sh, sem):
    c = lax.axis_index("c")
    s = lax.axis_index("s")
    _fill_rows(ones_v, CH, 16, 0.0)
    for off, sz in _ZCHUNKS:
        pltpu.sync_copy(ones_v.at[pl.ds(0, sz)],
                        hist_sh.at[pl.ds(s * RPT + off, sz)])
    _fill_rows(ones_v, CH, 16, 1.0)
    plsc.subcore_barrier()

    GRP = 16

    def do_slice(sl, carry):
        pltpu.sync_copy(se_hbm.at[c, sl], idx_v)

        def group(g, inner):
            def fire(j, x):
                pltpu.async_copy(ones_v, hist_sh.at[idx_v.at[g * GRP + j]],
                                 sem, add=True)
                return x

            lax.fori_loop(0, GRP, fire, 0)

            def drain(j, x):
                pltpu.make_async_copy(ones_v, hist_sh.at[idx_v.at[g * GRP + j]],
                                      sem).wait()
                return x

            lax.fori_loop(0, GRP, drain, 0)
            return inner

        lax.fori_loop(0, NCHUNK // GRP, group, 0)
        return carry

    lax.fori_loop(2 * s, 2 * s + 2, do_slice, 0)
    plsc.subcore_barrier()
    pltpu.sync_copy(hist_sh.at[pl.ds(s * RPT, RPT)],
                    deg_hbm.at[c, pl.ds(s * RPT, RPT)])


_deg_call = pl.kernel(
    _deg_body,
    out_type=jax.ShapeDtypeStruct((NC, NP, 16), jnp.float32),
    mesh=_mesh,
    compiler_params=_sc_params,
    scratch_types=[
        pltpu.VMEM((NCHUNK, CH), jnp.int32),
        pltpu.VMEM((CH, 16), jnp.float32),
        pltpu.VMEM_SHARED((NP, 16), jnp.float32),
        pltpu.SemaphoreType.DMA,
    ],
)


def _edge_body(K, t_hbm, se_hbm, out_hbm,
               sidx_v, didx_v, bufs, gsems, ssems, agg_sh):
    c = lax.axis_index("c")
    s = lax.axis_index("s")
    sl = c * NS + s

    pltpu.sync_copy(se_hbm.at[0, sl], sidx_v)
    pltpu.sync_copy(se_hbm.at[1, sl], didx_v)

    for k in range(K):
        _fill_rows(bufs[0], CH, D, 0.0)
        for off, sz in _ZCHUNKS:
            pltpu.sync_copy(bufs[0].at[pl.ds(0, sz)],
                            agg_sh.at[pl.ds(s * RPT + off, sz)])
        plsc.subcore_barrier()

        def gather(j, t):
            pltpu.async_copy(t_hbm.at[k].at[sidx_v.at[j]], bufs[t], gsems[t])

        def gwait(j, t):
            pltpu.make_async_copy(t_hbm.at[k].at[sidx_v.at[j]],
                                  bufs[t], gsems[t]).wait()

        def scat(j, t):
            pltpu.async_copy(bufs[t], agg_sh.at[didx_v.at[j]], ssems[t],
                             add=True)

        def swait(j, t):
            pltpu.make_async_copy(bufs[t], agg_sh.at[didx_v.at[j]],
                                  ssems[t]).wait()

        for t in range(NBUF):
            gather(t, t)

        def step(i, carry):
            j = NBUF * i
            for t in range(NBUF):
                gwait(j + t, t)
                scat(j + t, t)
            for t in range(NBUF):
                swait(j + t, t)

                @pl.when(j + t + NBUF < NCHUNK)
                def _():
                    gather(j + t + NBUF, t)

            return carry

        lax.fori_loop(0, NCHUNK // NBUF, step, 0)
        plsc.subcore_barrier()
        pltpu.sync_copy(agg_sh.at[pl.ds(s * RPT, RPT)],
                        out_hbm.at[c, k, pl.ds(s * RPT, RPT)])


def _make_edge_call(K):
    return pl.kernel(
        functools.partial(_edge_body, K),
        out_type=jax.ShapeDtypeStruct((NC, K, NP, D), jnp.float32),
        mesh=_mesh,
        compiler_params=_sc_params,
        scratch_types=[
            pltpu.VMEM((NCHUNK, CH), jnp.int32),
            pltpu.VMEM((NCHUNK, CH), jnp.int32),
            [pltpu.VMEM((CH, D), jnp.float32)] * NBUF,
            [pltpu.SemaphoreType.DMA] * NBUF,
            [pltpu.SemaphoreType.DMA] * NBUF,
            pltpu.VMEM_SHARED((NP, D), jnp.float32),
        ],
    )


_edge_call_l1 = _make_edge_call(2)
_edge_call_l2 = _make_edge_call(1)


NB = 1000


def _norms(deg_ref):
    i = pl.program_id(0)
    deg = deg_ref[:, pl.ds(i * NB, NB), :]
    nsrc = lax.rsqrt(jnp.maximum(deg[0, :, 0:1], 1.0))
    ndst = lax.rsqrt(jnp.maximum(deg[1, :, 0:1], 1.0))
    return nsrc, ndst


def _tc_mm1_body(feat_ref, w1_ref, xw_ref):
    xw_ref[...] = jnp.dot(feat_ref[...], w1_ref[...],
                          preferred_element_type=jnp.float32)


def _tc_scale_body(xw_ref, deg_ref, t1_ref):
    nsrc, _ = _norms(deg_ref)
    t1 = xw_ref[...] * nsrc
    t1_ref[...] = jnp.stack([t1[:, :D], t1[:, D:]])


def _tc_b_body(p_ref, deg_ref, w2_ref, t2_ref):
    nsrc, ndst = _norms(deg_ref)
    agg = p_ref[0] + p_ref[1]
    h1 = jnp.concatenate([agg[0], agg[1]], axis=1) * ndst
    h1 = jnp.maximum(h1, 0.0)
    t2_ref[...] = jnp.dot(h1 * nsrc, w2_ref[...],
                          preferred_element_type=jnp.float32)[None]


def _tc_c_body(p_ref, deg_ref, out_ref):
    _, ndst = _norms(deg_ref)
    out_ref[...] = (p_ref[0, 0] + p_ref[1, 0]) * ndst


_deg_full = pl.BlockSpec((2, NP, 16), lambda i: (0, 0, 0))


def _full(shape):
    return pl.BlockSpec(shape, lambda i: (0,) * len(shape))


def kernel(features, edge_index, W1, W2):
    se = edge_index.reshape(2, NT, NCHUNK, CH)

    xw = pl.pallas_call(
        _tc_mm1_body,
        grid=(N // NB,),
        in_specs=[pl.BlockSpec((NB, 128), lambda i: (i, 0)),
                  _full((128, 128))],
        out_specs=pl.BlockSpec((NB, 128), lambda i: (i, 0)),
        out_shape=jax.ShapeDtypeStruct((N, 128), jnp.float32),
    )(features, W1)

    deg2 = _deg_call(se)

    t1 = pl.pallas_call(
        _tc_scale_body,
        grid=(N // NB,),
        in_specs=[pl.BlockSpec((NB, 128), lambda i: (i, 0)),
                  _deg_full],
        out_specs=pl.BlockSpec((2, NB, D), lambda i: (0, i, 0)),
        out_shape=jax.ShapeDtypeStruct((2, N, D), jnp.float32),
    )(xw, deg2)

    p1 = _edge_call_l1(t1, se)

    t2 = pl.pallas_call(
        _tc_b_body,
        grid=(N // NB,),
        in_specs=[pl.BlockSpec((2, 2, NB, D), lambda i: (0, 0, i, 0)),
                  _deg_full,
                  _full((128, D))],
        out_specs=pl.BlockSpec((1, NB, D), lambda i: (0, i, 0)),
        out_shape=jax.ShapeDtypeStruct((1, N, D), jnp.float32),
    )(p1, deg2, W2)

    p2 = _edge_call_l2(t2, se)

    out = pl.pallas_call(
        _tc_c_body,
        grid=(N // NB,),
        in_specs=[pl.BlockSpec((2, 1, NB, D), lambda i: (0, 0, i, 0)),
                  _deg_full],
        out_specs=pl.BlockSpec((NB, D), lambda i: (i, 0)),
        out_shape=jax.ShapeDtypeStruct((N, D), jnp.float32),
    )(p2, deg2)

    return out


# speedup vs baseline: 1.0516x; 1.0516x over previous
import functools

import jax
import jax.numpy as jnp
from jax import lax
from jax.experimental import pallas as pl
from jax.experimental.pallas import tpu as pltpu
from jax.experimental.pallas import tpu_sc as plsc

N = 10000
NP = 10240
E = 320000
NC = 2
NS = 16
NT = NC * NS
CH = 125
NCHUNK = 80
RPT = NP // NS
D = 64
NBUF = 8

_mesh = plsc.VectorSubcoreMesh(
    core_axis_name="c", subcore_axis_name="s", num_cores=NC, num_subcores=NS)
_sc_params = pltpu.CompilerParams(use_tc_tiling_on_sc=False)

_ZCHUNKS = [(o, min(CH, RPT - o)) for o in range(0, RPT, CH)]


def _fill_rows(buf, rows, cols, value):
    vec = jnp.full((16,), value, jnp.float32)

    def body(r, carry):
        for k in range(cols // 16):
            buf[r, pl.ds(k * 16, 16)] = vec
        return carry

    lax.fori_loop(0, rows, body, 0)


def _deg_body(se_hbm, deg_hbm, idx_v, ones_v, hist_sh, sem):
    c = lax.axis_index("c")
    s = lax.axis_index("s")
    _fill_rows(ones_v, CH, 16, 0.0)
    for off, sz in _ZCHUNKS:
        pltpu.sync_copy(ones_v.at[pl.ds(0, sz)],
                        hist_sh.at[pl.ds(s * RPT + off, sz)])
    _fill_rows(ones_v, CH, 16, 1.0)
    plsc.subcore_barrier()

    GRP = 16

    def do_slice(sl, carry):
        pltpu.sync_copy(se_hbm.at[c, sl], idx_v)

        def group(g, inner):
            def fire(j, x):
                pltpu.async_copy(ones_v, hist_sh.at[idx_v.at[g * GRP + j]],
                                 sem, add=True)
                return x

            lax.fori_loop(0, GRP, fire, 0)

            def drain(j, x):
                pltpu.make_async_copy(ones_v, hist_sh.at[idx_v.at[g * GRP + j]],
                                      sem).wait()
                return x

            lax.fori_loop(0, GRP, drain, 0)
            return inner

        lax.fori_loop(0, NCHUNK // GRP, group, 0)
        return carry

    lax.fori_loop(2 * s, 2 * s + 2, do_slice, 0)
    plsc.subcore_barrier()
    pltpu.sync_copy(hist_sh.at[pl.ds(s * RPT, RPT)],
                    deg_hbm.at[c, pl.ds(s * RPT, RPT)])


_deg_call = pl.kernel(
    _deg_body,
    out_type=jax.ShapeDtypeStruct((NC, NP, 16), jnp.float32),
    mesh=_mesh,
    compiler_params=_sc_params,
    scratch_types=[
        pltpu.VMEM((NCHUNK, CH), jnp.int32),
        pltpu.VMEM((CH, 16), jnp.float32),
        pltpu.VMEM_SHARED((NP, 16), jnp.float32),
        pltpu.SemaphoreType.DMA,
    ],
)


def _edge_body(K, t_hbm, se_hbm, out_hbm,
               sidx_v, didx_v, bufs, gsems, ssems, agg_sh):
    c = lax.axis_index("c")
    s = lax.axis_index("s")
    sl = c * NS + s

    pltpu.sync_copy(se_hbm.at[0, sl], sidx_v)
    pltpu.sync_copy(se_hbm.at[1, sl], didx_v)

    for k in range(K):
        _fill_rows(bufs[0], CH, D, 0.0)
        for off, sz in _ZCHUNKS:
            pltpu.sync_copy(bufs[0].at[pl.ds(0, sz)],
                            agg_sh.at[pl.ds(s * RPT + off, sz)])
        plsc.subcore_barrier()

        def gather(j, t):
            pltpu.async_copy(t_hbm.at[k].at[sidx_v.at[j]], bufs[t], gsems[t])

        def gwait(j, t):
            pltpu.make_async_copy(t_hbm.at[k].at[sidx_v.at[j]],
                                  bufs[t], gsems[t]).wait()

        def scat(j, t):
            pltpu.async_copy(bufs[t], agg_sh.at[didx_v.at[j]], ssems[t],
                             add=True)

        def swait(j, t):
            pltpu.make_async_copy(bufs[t], agg_sh.at[didx_v.at[j]],
                                  ssems[t]).wait()

        for t in range(NBUF):
            gather(t, t)

        def step(i, carry):
            j = NBUF * i
            for t in range(NBUF):
                gwait(j + t, t)
                scat(j + t, t)
            for t in range(NBUF):
                swait(j + t, t)

                @pl.when(j + t + NBUF < NCHUNK)
                def _():
                    gather(j + t + NBUF, t)

            return carry

        lax.fori_loop(0, NCHUNK // NBUF, step, 0)
        plsc.subcore_barrier()
        pltpu.sync_copy(agg_sh.at[pl.ds(s * RPT, RPT)],
                        out_hbm.at[c, k, pl.ds(s * RPT, RPT)])


def _make_edge_call(K):
    return pl.kernel(
        functools.partial(_edge_body, K),
        out_type=jax.ShapeDtypeStruct((NC, K, NP, D), jnp.float32),
        mesh=_mesh,
        compiler_params=_sc_params,
        scratch_types=[
            pltpu.VMEM((NCHUNK, CH), jnp.int32),
            pltpu.VMEM((NCHUNK, CH), jnp.int32),
            [pltpu.VMEM((CH, D), jnp.float32)] * NBUF,
            [pltpu.SemaphoreType.DMA] * NBUF,
            [pltpu.SemaphoreType.DMA] * NBUF,
            pltpu.VMEM_SHARED((NP, D), jnp.float32),
        ],
    )


_edge_call_l1 = _make_edge_call(2)
_edge_call_l2 = _make_edge_call(1)


def _norms(deg_ref):
    deg = deg_ref[...]
    nsrc = lax.rsqrt(jnp.maximum(deg[0, :N, 0:1], 1.0))
    ndst = lax.rsqrt(jnp.maximum(deg[1, :N, 0:1], 1.0))
    return nsrc, ndst


def _tc_mm1_body(feat_ref, w1_ref, xw_ref):
    xw_ref[...] = jnp.dot(feat_ref[...], w1_ref[...],
                          preferred_element_type=jnp.float32)


def _tc_scale_body(xw_ref, deg_ref, t1_ref):
    nsrc, _ = _norms(deg_ref)
    t1 = xw_ref[...] * nsrc
    t1_ref[...] = jnp.stack([t1[:, :D], t1[:, D:]])


def _tc_b_body(p_ref, deg_ref, w2_ref, t2_ref):
    nsrc, ndst = _norms(deg_ref)
    agg = p_ref[0] + p_ref[1]
    h1 = jnp.concatenate([agg[0, :N], agg[1, :N]], axis=1) * ndst
    h1 = jnp.maximum(h1, 0.0)
    t2_ref[...] = jnp.dot(h1 * nsrc, w2_ref[...],
                          preferred_element_type=jnp.float32)[None]


def _tc_c_body(p_ref, deg_ref, out_ref):
    _, ndst = _norms(deg_ref)
    out_ref[...] = (p_ref[0, 0, :N] + p_ref[1, 0, :N]) * ndst


def kernel(features, edge_index, W1, W2):
    se = edge_index.reshape(2, NT, NCHUNK, CH)

    xw = pl.pallas_call(
        _tc_mm1_body,
        out_shape=jax.ShapeDtypeStruct((N, 128), jnp.float32),
    )(features, W1)

    deg2 = _deg_call(se)

    t1 = pl.pallas_call(
        _tc_scale_body,
        out_shape=jax.ShapeDtypeStruct((2, N, D), jnp.float32),
    )(xw, deg2)

    p1 = _edge_call_l1(t1, se)

    t2 = pl.pallas_call(
        _tc_b_body,
        out_shape=jax.ShapeDtypeStruct((1, N, D), jnp.float32),
    )(p1, deg2, W2)

    p2 = _edge_call_l2(t2, se)

    out = pl.pallas_call(
        _tc_c_body,
        out_shape=jax.ShapeDtypeStruct((N, D), jnp.float32),
    )(p2, deg2)

    return out


# speedup vs baseline: 1.0548x; 1.0031x over previous
import functools

import jax
import jax.numpy as jnp
from jax import lax
from jax.experimental import pallas as pl
from jax.experimental.pallas import tpu as pltpu
from jax.experimental.pallas import tpu_sc as plsc

N = 10000
NP = 10240
E = 320000
NC = 2
NS = 16
NT = NC * NS
CH = 125
NCHUNK = 80
RPT = NP // NS
D = 64
NBUF = 8

_mesh = plsc.VectorSubcoreMesh(
    core_axis_name="c", subcore_axis_name="s", num_cores=NC, num_subcores=NS)
_sc_params = pltpu.CompilerParams(use_tc_tiling_on_sc=False)

_ZCHUNKS = [(o, min(CH, RPT - o)) for o in range(0, RPT, CH)]


def _fill_rows(buf, rows, cols, value):
    vec = jnp.full((16,), value, jnp.float32)

    def body(r, carry):
        for k in range(cols // 16):
            buf[r, pl.ds(k * 16, 16)] = vec
        return carry

    lax.fori_loop(0, rows, body, 0)


def _deg_body(se_hbm, deg_hbm, idx_v, ones_v, hist_sh, sem):
    c = lax.axis_index("c")
    s = lax.axis_index("s")
    _fill_rows(ones_v, CH, 16, 0.0)
    for off, sz in _ZCHUNKS:
        pltpu.sync_copy(ones_v.at[pl.ds(0, sz)],
                        hist_sh.at[pl.ds(s * RPT + off, sz)])
    _fill_rows(ones_v, CH, 16, 1.0)
    plsc.subcore_barrier()

    GRP = 40

    def do_slice(sl, carry):
        pltpu.sync_copy(se_hbm.at[c, sl], idx_v)

        def group(g, inner):
            def fire(j, x):
                pltpu.async_copy(ones_v, hist_sh.at[idx_v.at[g * GRP + j]],
                                 sem, add=True)
                return x

            lax.fori_loop(0, GRP, fire, 0)

            def drain(j, x):
                pltpu.make_async_copy(ones_v, hist_sh.at[idx_v.at[g * GRP + j]],
                                      sem).wait()
                return x

            lax.fori_loop(0, GRP, drain, 0)
            return inner

        lax.fori_loop(0, NCHUNK // GRP, group, 0)
        return carry

    lax.fori_loop(2 * s, 2 * s + 2, do_slice, 0)
    plsc.subcore_barrier()
    pltpu.sync_copy(hist_sh.at[pl.ds(s * RPT, RPT)],
                    deg_hbm.at[c, pl.ds(s * RPT, RPT)])


_deg_call = pl.kernel(
    _deg_body,
    out_type=jax.ShapeDtypeStruct((NC, NP, 16), jnp.float32),
    mesh=_mesh,
    compiler_params=_sc_params,
    scratch_types=[
        pltpu.VMEM((NCHUNK, CH), jnp.int32),
        pltpu.VMEM((CH, 16), jnp.float32),
        pltpu.VMEM_SHARED((NP, 16), jnp.float32),
        pltpu.SemaphoreType.DMA,
    ],
)


def _edge_body(K, t_hbm, se_hbm, out_hbm,
               sidx_v, didx_v, bufs, gsems, ssems, agg_sh):
    c = lax.axis_index("c")
    s = lax.axis_index("s")
    sl = c * NS + s

    pltpu.sync_copy(se_hbm.at[0, sl], sidx_v)
    pltpu.sync_copy(se_hbm.at[1, sl], didx_v)

    for k in range(K):
        _fill_rows(bufs[0], CH, D, 0.0)
        for off, sz in _ZCHUNKS:
            pltpu.sync_copy(bufs[0].at[pl.ds(0, sz)],
                            agg_sh.at[pl.ds(s * RPT + off, sz)])
        plsc.subcore_barrier()

        def gather(j, t):
            pltpu.async_copy(t_hbm.at[k].at[sidx_v.at[j]], bufs[t], gsems[t])

        def gwait(j, t):
            pltpu.make_async_copy(t_hbm.at[k].at[sidx_v.at[j]],
                                  bufs[t], gsems[t]).wait()

        def scat(j, t):
            pltpu.async_copy(bufs[t], agg_sh.at[didx_v.at[j]], ssems[t],
                             add=True)

        def swait(j, t):
            pltpu.make_async_copy(bufs[t], agg_sh.at[didx_v.at[j]],
                                  ssems[t]).wait()

        for t in range(NBUF):
            gather(t, t)

        def step(i, carry):
            j = NBUF * i
            for t in range(NBUF):
                gwait(j + t, t)
                scat(j + t, t)
            for t in range(NBUF):
                swait(j + t, t)

                @pl.when(j + t + NBUF < NCHUNK)
                def _():
                    gather(j + t + NBUF, t)

            return carry

        lax.fori_loop(0, NCHUNK // NBUF, step, 0)
        plsc.subcore_barrier()
        pltpu.sync_copy(agg_sh.at[pl.ds(s * RPT, RPT)],
                        out_hbm.at[c, k, pl.ds(s * RPT, RPT)])


def _make_edge_call(K):
    return pl.kernel(
        functools.partial(_edge_body, K),
        out_type=jax.ShapeDtypeStruct((NC, K, NP, D), jnp.float32),
        mesh=_mesh,
        compiler_params=_sc_params,
        scratch_types=[
            pltpu.VMEM((NCHUNK, CH), jnp.int32),
            pltpu.VMEM((NCHUNK, CH), jnp.int32),
            [pltpu.VMEM((CH, D), jnp.float32)] * NBUF,
            [pltpu.SemaphoreType.DMA] * NBUF,
            [pltpu.SemaphoreType.DMA] * NBUF,
            pltpu.VMEM_SHARED((NP, D), jnp.float32),
        ],
    )


_edge_call_l1 = _make_edge_call(2)
_edge_call_l2 = _make_edge_call(1)


def _norms(deg_ref):
    deg = deg_ref[...]
    nsrc = lax.rsqrt(jnp.maximum(deg[0, :N, 0:1], 1.0))
    ndst = lax.rsqrt(jnp.maximum(deg[1, :N, 0:1], 1.0))
    return nsrc, ndst


def _tc_mm1_body(feat_ref, w1_ref, xw_ref):
    xw_ref[...] = jnp.dot(feat_ref[...], w1_ref[...],
                          preferred_element_type=jnp.float32)


def _tc_scale_body(xw_ref, deg_ref, t1_ref):
    nsrc, _ = _norms(deg_ref)
    t1 = xw_ref[...] * nsrc
    t1_ref[...] = jnp.stack([t1[:, :D], t1[:, D:]])


def _tc_b_body(p_ref, deg_ref, w2_ref, t2_ref):
    nsrc, ndst = _norms(deg_ref)
    agg = p_ref[0] + p_ref[1]
    h1 = jnp.concatenate([agg[0, :N], agg[1, :N]], axis=1) * ndst
    h1 = jnp.maximum(h1, 0.0)
    t2_ref[...] = jnp.dot(h1 * nsrc, w2_ref[...],
                          preferred_element_type=jnp.float32)[None]


def _tc_c_body(p_ref, deg_ref, out_ref):
    _, ndst = _norms(deg_ref)
    out_ref[...] = (p_ref[0, 0, :N] + p_ref[1, 0, :N]) * ndst


def kernel(features, edge_index, W1, W2):
    se = edge_index.reshape(2, NT, NCHUNK, CH)

    xw = pl.pallas_call(
        _tc_mm1_body,
        out_shape=jax.ShapeDtypeStruct((N, 128), jnp.float32),
    )(features, W1)

    deg2 = _deg_call(se)

    t1 = pl.pallas_call(
        _tc_scale_body,
        out_shape=jax.ShapeDtypeStruct((2, N, D), jnp.float32),
    )(xw, deg2)

    p1 = _edge_call_l1(t1, se)

    t2 = pl.pallas_call(
        _tc_b_body,
        out_shape=jax.ShapeDtypeStruct((1, N, D), jnp.float32),
    )(p1, deg2, W2)

    p2 = _edge_call_l2(t2, se)

    out = pl.pallas_call(
        _tc_c_body,
        out_shape=jax.ShapeDtypeStruct((N, D), jnp.float32),
    )(p2, deg2)

    return out
